# baseline (speedup 1.0000x reference)
import functools

import jax
import jax.numpy as jnp
from jax import lax
from jax.experimental import pallas as pl
from jax.experimental.pallas import tpu as pltpu
from jax.experimental.pallas import tpu_sc as plsc

N_REL_C = 4

CHUNK = 3200
UNROLL = 24


@functools.lru_cache(maxsize=None)
def _build(n_nodes, n_rel, dim, n_edges):
    info = plsc.get_sparse_core_info()
    nc, ns = info.num_cores, info.num_subcores
    nw = nc * ns
    assert dim % nw == 0
    dpw = dim // nw
    slice_words = n_nodes * dpw
    table_words = n_nodes * n_rel
    n_chunks = n_edges // CHUNK
    assert n_chunks * CHUNK == n_edges and n_chunks % 2 == 0
    ipc = CHUNK // 16

    mesh = plsc.VectorSubcoreMesh(core_axis_name="c", subcore_axis_name="s")

    @functools.partial(
        pl.kernel,
        out_type=jax.ShapeDtypeStruct((nw, slice_words), jnp.float32),
        mesh=mesh,
        compiler_params=pltpu.CompilerParams(needs_layout_passes=False),
        scratch_types=[
            pltpu.VMEM((slice_words // 2,), jnp.int32),
            pltpu.VMEM((slice_words,), jnp.float32),
            pltpu.VMEM((table_words,), jnp.float32),
            pltpu.VMEM((2, 3 * CHUNK), jnp.int32),
            pltpu.SemaphoreType.DMA,
            pltpu.SemaphoreType.DMA,
            pltpu.SemaphoreType.DMA,
        ],
    )
    def k(emb_hbm, edges_hbm, out_hbm,
          emb_v, out_v, cnt_v, ebuf, sem0, sem1, sem_e):
        wid = lax.axis_index("s") * nc + lax.axis_index("c")
        sems = (sem0, sem1)

        emb_cp = pltpu.async_copy(emb_hbm.at[wid], emb_v, sem_e)

        zf = jnp.zeros((16,), jnp.float32)

        @plsc.parallel_loop(0, slice_words // 16, unroll=UNROLL)
        def _(i):
            out_v[pl.ds(i * 16, 16)] = zf

        @plsc.parallel_loop(0, table_words // 16, unroll=UNROLL)
        def _(i):
            cnt_v[pl.ds(i * 16, 16)] = zf

        ones = jnp.ones((16,), jnp.float32)

        def streamed(compute, lo, sz):
            def start(g, slot):
                pltpu.async_copy(edges_hbm.at[g, pl.ds(lo, sz)],
                                 ebuf.at[slot, pl.ds(lo, sz)], sems[slot])

            def wait(g, slot):
                pltpu.make_async_copy(edges_hbm.at[g, pl.ds(lo, sz)],
                                      ebuf.at[slot, pl.ds(lo, sz)],
                                      sems[slot]).wait()

            start(0, 0)
            start(1, 1)

            def pair(g2, carry):
                g = g2 * 2
                wait(g, 0)
                compute(0)
                start(g + 2, 0)
                wait(g + 1, 1)
                compute(1)
                start(g + 3, 1)
                return carry

            lax.fori_loop(0, n_chunks // 2 - 1, pair, 0)
            wait(n_chunks - 2, 0)
            compute(0)
            wait(n_chunks - 1, 1)
            compute(1)

        def count_chunk(slot):
            @plsc.parallel_loop(0, ipc, unroll=UNROLL)
            def _(i):
                c16 = ebuf[slot, pl.ds(2 * CHUNK + i * 16, 16)]
                plsc.addupdate_scatter(cnt_v, [c16], ones)

        streamed(count_chunk, 2 * CHUNK, CHUNK)

        @plsc.parallel_loop(0, table_words // 16, unroll=UNROLL)
        def _(i):
            c16 = cnt_v[pl.ds(i * 16, 16)]
            cnt_v[pl.ds(i * 16, 16)] = 1.0 / jnp.maximum(c16, 1.0)

        emb_cp.wait()

        def accum_chunk(slot):
            @plsc.parallel_loop(0, ipc, unroll=UNROLL)
            def _(i):
                s16 = ebuf[slot, pl.ds(i * 16, 16)]
                d16 = ebuf[slot, pl.ds(CHUNK + i * 16, 16)]
                c16 = ebuf[slot, pl.ds(2 * CHUNK + i * 16, 16)]
                scale = plsc.load_gather(cnt_v, [c16])
                for p in range(dpw // 2):
                    w16 = plsc.load_gather(emb_v, [s16 + p * n_nodes])
                    a, b = plsc.unpack(plsc.bitcast(w16, jnp.bfloat16),
                                       format=plsc.PackFormat.INTERLEAVED)
                    plsc.addupdate_scatter(
                        out_v, [d16 + (2 * p) * n_nodes], a * scale)
                    plsc.addupdate_scatter(
                        out_v, [d16 + (2 * p + 1) * n_nodes], b * scale)

        streamed(accum_chunk, 0, 3 * CHUNK)

        pltpu.sync_copy(out_v, out_hbm.at[wid])

    return k, nw, dpw, n_chunks


def kernel(entity_emb, edge_index, edge_type):
    n_nodes, dim = entity_emb.shape
    n_edges = edge_type.shape[0]
    k, nw, dpw, n_chunks = _build(n_nodes, N_REL_C, dim, n_edges)
    pairs = jax.lax.bitcast_convert_type(
        entity_emb.astype(jnp.bfloat16).reshape(n_nodes, dim // 2, 2),
        jnp.int32)
    emb_t = pairs.transpose(1, 0).reshape(nw, (dpw // 2) * n_nodes)
    comb = edge_type * n_nodes + edge_index[1]
    edges = jnp.stack([edge_index[0].reshape(n_chunks, CHUNK),
                       edge_index[1].reshape(n_chunks, CHUNK),
                       comb.reshape(n_chunks, CHUNK)],
                      axis=1).reshape(n_chunks, 3 * CHUNK)
    out_t = k(emb_t, edges)
    return out_t.reshape(dim, n_nodes).transpose(1, 0)


# speedup vs baseline: 1.2514x; 1.2514x over previous
import functools

import jax
import jax.numpy as jnp
from jax import lax
from jax.experimental import pallas as pl
from jax.experimental.pallas import tpu as pltpu
from jax.experimental.pallas import tpu_sc as plsc

N_REL_C = 4

CHUNK = 3200
UNROLL = 16


@functools.lru_cache(maxsize=None)
def _build(n_nodes, n_rel, dim, n_edges):
    info = plsc.get_sparse_core_info()
    nc, ns = info.num_cores, info.num_subcores
    nw = nc * ns
    assert dim % nw == 0
    dpw = dim // nw
    slice_words = n_nodes * dpw
    table_words = n_nodes * n_rel
    n_chunks = n_edges // CHUNK
    assert n_chunks * CHUNK == n_edges and n_chunks % 2 == 0
    ipc = CHUNK // 16

    mesh = plsc.VectorSubcoreMesh(core_axis_name="c", subcore_axis_name="s")

    @functools.partial(
        pl.kernel,
        out_type=jax.ShapeDtypeStruct((nw, slice_words), jnp.float32),
        mesh=mesh,
        compiler_params=pltpu.CompilerParams(needs_layout_passes=False),
        scratch_types=[
            pltpu.VMEM((slice_words // 2,), jnp.int32),
            pltpu.VMEM((slice_words,), jnp.float32),
            pltpu.VMEM((table_words,), jnp.float32),
            pltpu.VMEM((2, 3 * CHUNK), jnp.int32),
            pltpu.SemaphoreType.DMA,
            pltpu.SemaphoreType.DMA,
            pltpu.SemaphoreType.DMA,
        ],
    )
    def k(emb_hbm, edges_hbm, out_hbm,
          emb_v, out_v, cnt_v, ebuf, sem0, sem1, sem_e):
        wid = lax.axis_index("s") * nc + lax.axis_index("c")
        sems = (sem0, sem1)

        emb_cp = pltpu.async_copy(emb_hbm.at[wid], emb_v, sem_e)

        zf = jnp.zeros((16,), jnp.float32)

        @plsc.parallel_loop(0, slice_words // 16, unroll=UNROLL)
        def _(i):
            out_v[pl.ds(i * 16, 16)] = zf

        @plsc.parallel_loop(0, table_words // 16, unroll=UNROLL)
        def _(i):
            cnt_v[pl.ds(i * 16, 16)] = zf

        ones = jnp.ones((16,), jnp.float32)

        def streamed(compute, lo, sz):
            def start(g, slot):
                pltpu.async_copy(edges_hbm.at[g, pl.ds(lo, sz)],
                                 ebuf.at[slot, pl.ds(lo, sz)], sems[slot])

            def wait(g, slot):
                pltpu.make_async_copy(edges_hbm.at[g, pl.ds(lo, sz)],
                                      ebuf.at[slot, pl.ds(lo, sz)],
                                      sems[slot]).wait()

            start(0, 0)
            start(1, 1)

            def pair(g2, carry):
                g = g2 * 2
                wait(g, 0)
                compute(0)
                start(g + 2, 0)
                wait(g + 1, 1)
                compute(1)
                start(g + 3, 1)
                return carry

            lax.fori_loop(0, n_chunks // 2 - 1, pair, 0)
            wait(n_chunks - 2, 0)
            compute(0)
            wait(n_chunks - 1, 1)
            compute(1)

        def count_chunk(slot):
            @plsc.parallel_loop(0, ipc, unroll=UNROLL)
            def _(i):
                c16 = ebuf[slot, pl.ds(2 * CHUNK + i * 16, 16)]
                plsc.addupdate_scatter(cnt_v, [c16], ones)

        streamed(count_chunk, 2 * CHUNK, CHUNK)

        @plsc.parallel_loop(0, table_words // 16, unroll=UNROLL)
        def _(i):
            c16 = cnt_v[pl.ds(i * 16, 16)]
            cnt_v[pl.ds(i * 16, 16)] = 1.0 / jnp.maximum(c16, 1.0)

        emb_cp.wait()

        def accum_chunk(slot):
            @plsc.parallel_loop(0, ipc, unroll=UNROLL)
            def _(i):
                s16 = ebuf[slot, pl.ds(i * 16, 16)]
                d16 = ebuf[slot, pl.ds(CHUNK + i * 16, 16)]
                c16 = ebuf[slot, pl.ds(2 * CHUNK + i * 16, 16)]
                scale = plsc.load_gather(cnt_v, [c16])
                for p in range(dpw // 2):
                    w16 = plsc.load_gather(emb_v, [s16 + p * n_nodes])
                    a, b = plsc.unpack(plsc.bitcast(w16, jnp.bfloat16),
                                       format=plsc.PackFormat.INTERLEAVED)
                    plsc.addupdate_scatter(
                        out_v, [d16 + (2 * p) * n_nodes], a * scale)
                    plsc.addupdate_scatter(
                        out_v, [d16 + (2 * p + 1) * n_nodes], b * scale)

        streamed(accum_chunk, 0, 3 * CHUNK)

        pltpu.sync_copy(out_v, out_hbm.at[wid])

    return k, nw, dpw, n_chunks


def kernel(entity_emb, edge_index, edge_type):
    n_nodes, dim = entity_emb.shape
    n_edges = edge_type.shape[0]
    k, nw, dpw, n_chunks = _build(n_nodes, N_REL_C, dim, n_edges)
    pairs = jax.lax.bitcast_convert_type(
        entity_emb.astype(jnp.bfloat16).reshape(n_nodes, dim // 2, 2),
        jnp.int32)
    emb_t = pairs.transpose(1, 0).reshape(nw, (dpw // 2) * n_nodes)
    comb = edge_type * n_nodes + edge_index[1]
    edges = jnp.stack([edge_index[0].reshape(n_chunks, CHUNK),
                       edge_index[1].reshape(n_chunks, CHUNK),
                       comb.reshape(n_chunks, CHUNK)],
                      axis=1).reshape(n_chunks, 3 * CHUNK)
    out_t = k(emb_t, edges)
    return out_t.reshape(dim, n_nodes).transpose(1, 0)
